# speedup vs baseline: 1.6431x; 1.6431x over previous
import functools

import jax
import jax.numpy as jnp
from jax import lax
from jax.experimental import pallas as pl
from jax.experimental.pallas import tpu as pltpu
from jax.experimental.pallas import tpu_sc as plsc

VOCAB = 100000
D = 2048
B_TOKENS = 4 * 2048

_info = plsc.get_sparse_core_info()
NC = _info.num_cores
NS = _info.num_subcores
NW = NC * NS
ROWS_PER_W = B_TOKENS // NW
K = 16
NCHUNK = ROWS_PER_W // K
NBUF = 2

_mesh = plsc.VectorSubcoreMesh(core_axis_name="c", subcore_axis_name="s")


@functools.partial(
    pl.kernel,
    mesh=_mesh,
    out_type=jax.ShapeDtypeStruct((B_TOKENS, D), jnp.float32),
    scratch_types=[
        pltpu.VMEM((NCHUNK, K), jnp.int32),
        pltpu.VMEM((K, D), jnp.float32),
        pltpu.VMEM((K, D), jnp.float32),
        pltpu.SemaphoreType.DMA,
        pltpu.SemaphoreType.DMA,
        pltpu.SemaphoreType.DMA,
        pltpu.SemaphoreType.DMA,
    ],
)
def _gather_rows(table_hbm, idx_hbm, out_hbm, idx_v, buf0, buf1,
                 gsem0, gsem1, wsem0, wsem1):
    wid = lax.axis_index("s") * NC + lax.axis_index("c")
    base = wid * ROWS_PER_W
    pltpu.sync_copy(idx_hbm.at[wid], idx_v)

    bufs = (buf0, buf1)
    gsems = (gsem0, gsem1)
    wsems = (wsem0, wsem1)

    for j in range(NBUF):
        pltpu.async_copy(table_hbm.at[idx_v.at[j]], bufs[j], gsems[j])

    for j in range(NCHUNK):
        b = j % NBUF
        pltpu.make_async_copy(table_hbm.at[idx_v.at[j]], bufs[b],
                              gsems[b]).wait()
        pltpu.async_copy(bufs[b], out_hbm.at[pl.ds(base + j * K, K)],
                         wsems[b])
        if j + NBUF < NCHUNK:
            pltpu.make_async_copy(
                bufs[b], out_hbm.at[pl.ds(base + j * K, K)], wsems[b]).wait()
            pltpu.async_copy(table_hbm.at[idx_v.at[j + NBUF]], bufs[b],
                             gsems[b])

    for j in range(NCHUNK - NBUF, NCHUNK):
        b = j % NBUF
        pltpu.make_async_copy(bufs[b], out_hbm.at[pl.ds(base + j * K, K)],
                              wsems[b]).wait()


def kernel(prompt_completion_ids, attention_mask, weight):
    ids = prompt_completion_ids.reshape(NW, NCHUNK, K)
    flat = _gather_rows(weight, ids)
    hidden_states = flat.reshape(prompt_completion_ids.shape + (D,))
    return (hidden_states, attention_mask)


# speedup vs baseline: 1.6565x; 1.0082x over previous
import functools

import jax
import jax.numpy as jnp
from jax import lax
from jax.experimental import pallas as pl
from jax.experimental.pallas import tpu as pltpu
from jax.experimental.pallas import tpu_sc as plsc

VOCAB = 100000
D = 2048
B_TOKENS = 4 * 2048

_info = plsc.get_sparse_core_info()
NC = _info.num_cores
NS = _info.num_subcores
NW = NC * NS
ROWS_PER_W = B_TOKENS // NW
K = 16
NCHUNK = ROWS_PER_W // K
NBUF = 3

_mesh = plsc.VectorSubcoreMesh(core_axis_name="c", subcore_axis_name="s")


@functools.partial(
    pl.kernel,
    mesh=_mesh,
    out_type=jax.ShapeDtypeStruct((B_TOKENS, D), jnp.float32),
    scratch_types=[
        pltpu.VMEM((NCHUNK, K), jnp.int32),
        pltpu.VMEM((K, D), jnp.float32),
        pltpu.VMEM((K, D), jnp.float32),
        pltpu.VMEM((K, D), jnp.float32),
        pltpu.SemaphoreType.DMA,
        pltpu.SemaphoreType.DMA,
        pltpu.SemaphoreType.DMA,
        pltpu.SemaphoreType.DMA,
        pltpu.SemaphoreType.DMA,
        pltpu.SemaphoreType.DMA,
    ],
)
def _gather_rows(table_hbm, idx_hbm, out_hbm, idx_v, buf0, buf1, buf2,
                 gsem0, gsem1, gsem2, wsem0, wsem1, wsem2):
    wid = lax.axis_index("s") * NC + lax.axis_index("c")
    base = wid * ROWS_PER_W
    pltpu.sync_copy(idx_hbm.at[wid], idx_v)

    bufs = (buf0, buf1, buf2)
    gsems = (gsem0, gsem1, gsem2)
    wsems = (wsem0, wsem1, wsem2)

    for j in range(NBUF):
        pltpu.async_copy(table_hbm.at[idx_v.at[j]], bufs[j], gsems[j])

    for j in range(NCHUNK):
        b = j % NBUF
        pltpu.make_async_copy(table_hbm.at[idx_v.at[j]], bufs[b],
                              gsems[b]).wait()
        pltpu.async_copy(bufs[b], out_hbm.at[pl.ds(base + j * K, K)],
                         wsems[b])
        if j + NBUF < NCHUNK:
            pltpu.make_async_copy(
                bufs[b], out_hbm.at[pl.ds(base + j * K, K)], wsems[b]).wait()
            pltpu.async_copy(table_hbm.at[idx_v.at[j + NBUF]], bufs[b],
                             gsems[b])

    for j in range(NCHUNK - NBUF, NCHUNK):
        b = j % NBUF
        pltpu.make_async_copy(bufs[b], out_hbm.at[pl.ds(base + j * K, K)],
                              wsems[b]).wait()


def kernel(prompt_completion_ids, attention_mask, weight):
    ids = prompt_completion_ids.reshape(NW, NCHUNK, K)
    flat = _gather_rows(weight, ids)
    hidden_states = flat.reshape(prompt_completion_ids.shape + (D,))
    return (hidden_states, attention_mask)
